# baseline (speedup 1.0000x reference)
import jax
import jax.numpy as jnp
from jax import lax
from jax.experimental import pallas as pl
from jax.experimental.pallas import tpu as pltpu
from jax.experimental.pallas import tpu_sc as plsc

NUM_CORES = 2
NUM_SUBCORES = 16
NUM_WORKERS = NUM_CORES * NUM_SUBCORES
LANES = 16
CHUNK = 256


def _glove_body(pidx0_hbm, pidx1_hbm, c0_hbm, c1_hbm, par0_hbm, par1_hbm,
                w0_hbm, w1_hbm, b0_hbm, b1_hbm, out_hbm,
                pidx0_v, pidx1_v, c0_v, c1_v, par0_v, par1_v,
                e0_v, e1_v, bb0_v, bb1_v, out_v, sem):
    b_per_w = out_v.shape[0]
    dim = e0_v.shape[1] // 2
    wid = lax.axis_index("s") * NUM_CORES + lax.axis_index("c")
    base = wid * b_per_w

    for c in range(b_per_w // CHUNK):
        cbase = base + c * CHUNK
        pltpu.sync_copy(pidx0_hbm.at[pl.ds(cbase, CHUNK)], pidx0_v)
        pltpu.sync_copy(pidx1_hbm.at[pl.ds(cbase, CHUNK)], pidx1_v)
        pltpu.sync_copy(c0_hbm.at[pl.ds(cbase, CHUNK)], c0_v)
        pltpu.sync_copy(c1_hbm.at[pl.ds(cbase, CHUNK)], c1_v)
        pltpu.sync_copy(par0_hbm.at[pl.ds(cbase, CHUNK)], par0_v)
        pltpu.sync_copy(par1_hbm.at[pl.ds(cbase, CHUNK)], par1_v)
        copies = [
            pltpu.async_copy(w0_hbm.at[pidx0_v], e0_v, sem),
            pltpu.async_copy(w1_hbm.at[pidx1_v], e1_v, sem),
            pltpu.async_copy(b0_hbm.at[c0_v], bb0_v, sem),
            pltpu.async_copy(b1_hbm.at[c1_v], bb1_v, sem),
        ]
        for cp in copies:
            cp.wait()

        def grp_body(g, carry):
            s = g * LANES
            rows = s + lax.iota(jnp.int32, LANES)
            p0 = par0_v[pl.ds(s, LANES)]
            p1 = par1_v[pl.ds(s, LANES)]
            acc = bb0_v[pl.ds(s, LANES)] + bb1_v[pl.ds(s, LANES)]
            for d in range(dim):
                acc = acc + plsc.load_gather(e0_v, [rows, p0 + d]) * \
                    plsc.load_gather(e1_v, [rows, p1 + d])
            out_v[pl.ds(c * CHUNK + s, LANES)] = acc
            return carry

        lax.fori_loop(0, CHUNK // LANES, grp_body, 0)

    pltpu.sync_copy(out_v, out_hbm.at[pl.ds(base, b_per_w)])


def kernel(x, W0, W1, b0, b1):
    batch = x.shape[0]
    vocab, dim = W0.shape
    b_per_w = batch // NUM_WORKERS
    codes0 = x[:, 0].astype(jnp.int32)
    codes1 = x[:, 1].astype(jnp.int32)
    pidx0 = codes0 >> 1
    pidx1 = codes1 >> 1
    par0 = (codes0 & 1) * dim
    par1 = (codes1 & 1) * dim
    w0p = W0.reshape(vocab // 2, 2 * dim)
    w1p = W1.reshape(vocab // 2, 2 * dim)
    b0v = b0.reshape(-1)
    b1v = b1.reshape(-1)

    mesh = plsc.VectorSubcoreMesh(core_axis_name="c", subcore_axis_name="s")
    run = pl.kernel(
        _glove_body,
        out_type=jax.ShapeDtypeStruct((batch,), jnp.float32),
        mesh=mesh,
        compiler_params=pltpu.CompilerParams(needs_layout_passes=False),
        scratch_types=[
            pltpu.VMEM((CHUNK,), jnp.int32),
            pltpu.VMEM((CHUNK,), jnp.int32),
            pltpu.VMEM((CHUNK,), jnp.int32),
            pltpu.VMEM((CHUNK,), jnp.int32),
            pltpu.VMEM((CHUNK,), jnp.int32),
            pltpu.VMEM((CHUNK,), jnp.int32),
            pltpu.VMEM((CHUNK, 2 * dim), jnp.float32),
            pltpu.VMEM((CHUNK, 2 * dim), jnp.float32),
            pltpu.VMEM((CHUNK,), jnp.float32),
            pltpu.VMEM((CHUNK,), jnp.float32),
            pltpu.VMEM((b_per_w,), jnp.float32),
            pltpu.SemaphoreType.DMA,
        ],
    )
    return run(pidx0, pidx1, codes0, codes1, par0, par1, w0p, w1p, b0v, b1v)
